# baseline (speedup 1.0000x reference)
import functools

import jax
import jax.numpy as jnp
from jax import lax
from jax.experimental import pallas as pl
from jax.experimental.pallas import tpu as pltpu
from jax.experimental.pallas import tpu_sc as plsc

NC, NS = 2, 16
NW = NC * NS
B = 16384 * 26
D = 128
BPW = B // NW
CH = 128
NCHUNK = BPW // CH
GRP = 4
NG = NCHUNK // GRP


def _sc_body(idx_hbm, table_hbm, out_hbm, idx_v, rows_v, sem):
    wid = lax.axis_index("s") * NC + lax.axis_index("c")
    pltpu.sync_copy(idx_hbm.at[wid], idx_v)
    chunk_base = wid * NCHUNK

    def grp_body(g, carry):
        handles = []
        for b in range(GRP):
            j = g * GRP + b
            handles.append(
                pltpu.async_copy(table_hbm.at[idx_v.at[j]], rows_v.at[b], sem))
        for h in handles:
            h.wait()
        pltpu.sync_copy(rows_v, out_hbm.at[pl.ds(chunk_base + g * GRP, GRP)])
        return carry

    lax.fori_loop(0, NG, grp_body, 0)


@jax.jit
def kernel(indices, table):
    idx3 = indices.astype(jnp.int32).reshape(NW, NCHUNK, CH)
    mesh = plsc.VectorSubcoreMesh(core_axis_name="c", subcore_axis_name="s")
    k = functools.partial(
        pl.kernel,
        out_type=jax.ShapeDtypeStruct((B // CH, CH, D), jnp.float32),
        mesh=mesh,
        scratch_types=[
            pltpu.VMEM((NCHUNK, CH), jnp.int32),
            pltpu.VMEM((GRP, CH, D), jnp.float32),
            pltpu.SemaphoreType.DMA,
        ],
    )(_sc_body)
    out = k(idx3, table)
    return out.reshape(16384, 26, D)
